# baseline (speedup 1.0000x reference)
import functools
import math

import jax
import jax.numpy as jnp
from jax import lax
from jax.experimental import pallas as pl
from jax.experimental.pallas import tpu as pltpu

B, S, D, H = 1, 2048, 1024, 16
V, E, TOPK = 2, 8, 2
DFF, DFFG = 1024, 2048
DH = D // H

BT = 256
BQ = 256


def _qkv_body(x_ref, w_ref, b_ref, o_ref):
    o_ref[...] = (
        jnp.dot(x_ref[...], w_ref[...], preferred_element_type=jnp.float32)
        + b_ref[...]
    )


def _attn_body(q_ref, k_ref, v_ref, o_ref):
    si = pl.program_id(1)
    q = q_ref[0, :, :]
    k = k_ref[0, :, :]
    s = lax.dot_general(
        q, k, (((1,), (1,)), ((), ())), preferred_element_type=jnp.float32
    ) / math.sqrt(DH)
    rows = si * BQ + lax.broadcasted_iota(jnp.int32, (BQ, S), 0)
    cols = lax.broadcasted_iota(jnp.int32, (BQ, S), 1)
    s = jnp.where(cols > rows, jnp.float32(-1e9), s)
    p = jax.nn.softmax(s, axis=-1)
    o_ref[0, :, :] = jnp.dot(p, v_ref[0, :, :], preferred_element_type=jnp.float32)


def _oproj_ln_body(o_ref, w_ref, b_ref, x_ref, g_ref, beta_ref, out_ref):
    y = (
        jnp.dot(o_ref[...], w_ref[...], preferred_element_type=jnp.float32)
        + b_ref[...]
        + x_ref[...]
    )
    m = jnp.mean(y, axis=-1, keepdims=True)
    v = jnp.mean((y - m) ** 2, axis=-1, keepdims=True)
    out_ref[...] = (y - m) * lax.rsqrt(v + 1e-5) * g_ref[...] + beta_ref[...]


def _route_body(lg_ref, mk_ref, gates_ref, guide_ref):
    lg = lg_ref[...]
    probs = jax.nn.softmax(lg, axis=-1)
    iota_e = lax.broadcasted_iota(jnp.int32, (V * S, E), 1)
    m1 = jnp.max(probs, axis=-1, keepdims=True)
    i1 = jnp.min(jnp.where(probs == m1, iota_e, E), axis=-1, keepdims=True)
    oh1 = iota_e == i1
    p2 = jnp.where(oh1, jnp.float32(-1.0), probs)
    m2 = jnp.max(p2, axis=-1, keepdims=True)
    i2 = jnp.min(jnp.where(p2 == m2, iota_e, E), axis=-1, keepdims=True)
    oh2 = iota_e == i2
    ssum = m1 + m2
    gates = jnp.where(oh1, m1 / ssum, 0.0) + jnp.where(oh2, m2 / ssum, 0.0)
    gates_ref[...] = gates
    mk = mk_ref[...]
    mn = mk / (jnp.sum(mk, axis=-1, keepdims=True) + 1e-9)
    guide_ref[...] = (-jnp.sum(mn * jnp.log(probs + 1e-9)) / (S * V)).reshape(1, 1)


def _expert_body(x_ref, w1_ref, b1_ref, w2_ref, b2_ref, g_ref, eo_ref):
    ve = pl.program_id(0)
    x = x_ref[...]
    h = jax.nn.gelu(
        jnp.dot(x, w1_ref[0, :, :], preferred_element_type=jnp.float32)
        + b1_ref[0, :, :]
    )
    eo = (
        jnp.dot(h, w2_ref[0, :, :], preferred_element_type=jnp.float32)
        + b2_ref[0, :, :]
    )
    iota_ve = lax.broadcasted_iota(jnp.int32, (BT, V * E), 1)
    col = jnp.sum(
        jnp.where(iota_ve == ve, g_ref[...], 0.0), axis=1, keepdims=True
    )
    eo_ref[0, :, :] = eo * col


def _final_body(
    x_ref, eo_ref, w1_ref, b1_ref, w2_ref, b2_ref, x1_ref, g_ref, beta_ref, out_ref
):
    x = x_ref[...]
    h = jax.nn.gelu(
        jnp.dot(x, w1_ref[...], preferred_element_type=jnp.float32) + b1_ref[...]
    )
    gen = jnp.dot(h, w2_ref[...], preferred_element_type=jnp.float32) + b2_ref[...]
    fin = jnp.sum(eo_ref[...], axis=0) + gen + x1_ref[...]
    m = jnp.mean(fin, axis=-1, keepdims=True)
    v = jnp.mean((fin - m) ** 2, axis=-1, keepdims=True)
    out_ref[...] = (fin - m) * lax.rsqrt(v + 1e-5) * g_ref[...] + beta_ref[...]


def kernel(x, total_logits, total_masks, attn_mask, Wq, bq, Wk, bk, Wv, bv, Wo, bo,
           g1, beta1, g2, beta2, W1v, b1v, W2v, b2v, W1g, b1g, W2g, b2g):
    f32 = jnp.float32
    xf = x.reshape(S, D)

    Wqkv = jnp.concatenate([Wq, Wk, Wv], axis=1)
    bqkv = jnp.concatenate([bq, bk, bv]).reshape(1, 3 * D)
    qkv = pl.pallas_call(
        _qkv_body,
        grid=(S // BT,),
        in_specs=[
            pl.BlockSpec((BT, D), lambda i: (i, 0)),
            pl.BlockSpec((D, 3 * D), lambda i: (0, 0)),
            pl.BlockSpec((1, 3 * D), lambda i: (0, 0)),
        ],
        out_specs=pl.BlockSpec((BT, 3 * D), lambda i: (i, 0)),
        out_shape=jax.ShapeDtypeStruct((S, 3 * D), f32),
    )(xf, Wqkv, bqkv)

    q = qkv[:, :D].reshape(S, H, DH).transpose(1, 0, 2)
    k = qkv[:, D:2 * D].reshape(S, H, DH).transpose(1, 0, 2)
    v = qkv[:, 2 * D:].reshape(S, H, DH).transpose(1, 0, 2)

    o = pl.pallas_call(
        _attn_body,
        grid=(H, S // BQ),
        in_specs=[
            pl.BlockSpec((1, BQ, DH), lambda h, i: (h, i, 0)),
            pl.BlockSpec((1, S, DH), lambda h, i: (h, 0, 0)),
            pl.BlockSpec((1, S, DH), lambda h, i: (h, 0, 0)),
        ],
        out_specs=pl.BlockSpec((1, BQ, DH), lambda h, i: (h, i, 0)),
        out_shape=jax.ShapeDtypeStruct((H, S, DH), f32),
    )(q, k, v)
    o2 = o.transpose(1, 0, 2).reshape(S, D)

    x1 = pl.pallas_call(
        _oproj_ln_body,
        grid=(S // BT,),
        in_specs=[
            pl.BlockSpec((BT, D), lambda i: (i, 0)),
            pl.BlockSpec((D, D), lambda i: (0, 0)),
            pl.BlockSpec((1, D), lambda i: (0, 0)),
            pl.BlockSpec((BT, D), lambda i: (i, 0)),
            pl.BlockSpec((1, D), lambda i: (0, 0)),
            pl.BlockSpec((1, D), lambda i: (0, 0)),
        ],
        out_specs=pl.BlockSpec((BT, D), lambda i: (i, 0)),
        out_shape=jax.ShapeDtypeStruct((S, D), f32),
    )(o2, Wo, bo.reshape(1, D), xf, g1.reshape(1, D), beta1.reshape(1, D))

    lg = total_logits.reshape(V * S, E)
    mk = total_masks.reshape(V * S, E)
    gates_vs, guide2 = pl.pallas_call(
        _route_body,
        in_specs=[
            pl.BlockSpec((V * S, E), lambda: (0, 0)),
            pl.BlockSpec((V * S, E), lambda: (0, 0)),
        ],
        out_specs=[
            pl.BlockSpec((V * S, E), lambda: (0, 0)),
            pl.BlockSpec((1, 1), lambda: (0, 0)),
        ],
        out_shape=[
            jax.ShapeDtypeStruct((V * S, E), f32),
            jax.ShapeDtypeStruct((1, 1), f32),
        ],
    )(lg, mk)
    total_guide = guide2[0, 0]
    gates16 = gates_vs.reshape(V, S, E).transpose(1, 0, 2).reshape(S, V * E)

    W1r = W1v.reshape(V * E, D, DFF)
    b1r = b1v.reshape(V * E, 1, DFF)
    W2r = W2v.reshape(V * E, DFF, D)
    b2r = b2v.reshape(V * E, 1, D)
    eo_all = pl.pallas_call(
        _expert_body,
        grid=(V * E, S // BT),
        in_specs=[
            pl.BlockSpec((BT, D), lambda ve, i: (i, 0)),
            pl.BlockSpec((1, D, DFF), lambda ve, i: (ve, 0, 0)),
            pl.BlockSpec((1, 1, DFF), lambda ve, i: (ve, 0, 0)),
            pl.BlockSpec((1, DFF, D), lambda ve, i: (ve, 0, 0)),
            pl.BlockSpec((1, 1, D), lambda ve, i: (ve, 0, 0)),
            pl.BlockSpec((BT, V * E), lambda ve, i: (i, 0)),
        ],
        out_specs=pl.BlockSpec((1, BT, D), lambda ve, i: (ve, i, 0)),
        out_shape=jax.ShapeDtypeStruct((V * E, S, D), f32),
    )(x1, W1r, b1r, W2r, b2r, gates16)

    out = pl.pallas_call(
        _final_body,
        grid=(S // BT,),
        in_specs=[
            pl.BlockSpec((BT, D), lambda i: (i, 0)),
            pl.BlockSpec((V * E, BT, D), lambda i: (0, i, 0)),
            pl.BlockSpec((D, DFFG), lambda i: (0, 0)),
            pl.BlockSpec((1, DFFG), lambda i: (0, 0)),
            pl.BlockSpec((DFFG, D), lambda i: (0, 0)),
            pl.BlockSpec((1, D), lambda i: (0, 0)),
            pl.BlockSpec((BT, D), lambda i: (i, 0)),
            pl.BlockSpec((1, D), lambda i: (0, 0)),
            pl.BlockSpec((1, D), lambda i: (0, 0)),
        ],
        out_specs=pl.BlockSpec((BT, D), lambda i: (i, 0)),
        out_shape=jax.ShapeDtypeStruct((S, D), f32),
    )(
        x1, eo_all, W1g, b1g.reshape(1, DFFG), W2g, b2g.reshape(1, D),
        x1, g2.reshape(1, D), beta2.reshape(1, D),
    )

    return out.reshape(B, S, D), total_guide
